# Initial kernel scaffold; baseline (speedup 1.0000x reference)
import jax
import jax.numpy as jnp
from jax.experimental import pallas as pl


def kernel(signal, bc, norm_mean, norm_var, W_d0, b_d0, g_d0, be_d0, W_d1, b_d1, g_d1, be_d1, W_m, b_m, g_m, be_m, W_u0, b_u0, g_u0, be_u0, W_u1, b_u1, g_u1, be_u1, W_out, b_out):
    raise NotImplementedError("write your pallas kernel here")


# speedup vs baseline: 3.9942x; 3.9942x over previous
import functools

import jax
import jax.numpy as jnp
from jax import lax
from jax.experimental import pallas as pl
from jax.experimental.pallas import tpu as pltpu
from jax.experimental.pallas import tpu_sc as plsc

V = 6890
R = 5
A = 8
VPAD = 6912
J = V * R * A
JPAD = VPAD * R * A
NC, NS = 2, 16
NW = NC * NS
JW = JPAD // NW
BJ = 48
NBLK = JW // BJ
VB = 128


@functools.cache
def _interp_kernel(d):
    mesh = plsc.VectorSubcoreMesh(
        core_axis_name="c", subcore_axis_name="s", num_cores=NC, num_subcores=NS
    )
    nch = d // 16

    @functools.partial(
        pl.kernel,
        out_type=jax.ShapeDtypeStruct((JPAD, d), jnp.float32),
        mesh=mesh,
        compiler_params=pltpu.CompilerParams(use_tc_tiling_on_sc=False),
        scratch_types=[
            pltpu.VMEM((3 * JW,), jnp.int32),
            pltpu.VMEM((3 * JW + 16,), jnp.float32),
            pltpu.VMEM((3 * BJ, d), jnp.float32),
            pltpu.VMEM((BJ, d), jnp.float32),
            pltpu.SemaphoreType.DMA,
        ],
    )
    def kern(sig_hbm, idx_hbm, w_hbm, out_hbm, idx_v, w_v, rows_v, o_v, sem):
        wid = lax.axis_index("s") * NC + lax.axis_index("c")
        jb = wid * JW
        pltpu.sync_copy(idx_hbm.at[pl.ds(3 * jb, 3 * JW)], idx_v)
        pltpu.sync_copy(w_hbm.at[pl.ds(3 * jb, 3 * JW)], w_v.at[pl.ds(0, 3 * JW)])
        dn = lax.GatherDimensionNumbers(
            offset_dims=(), collapsed_slice_dims=(0,), start_index_map=(0,)
        )
        splat = functools.partial(
            lax.gather,
            dimension_numbers=dn,
            slice_sizes=(1,),
            mode=lax.GatherScatterMode.PROMISE_IN_BOUNDS,
        )

        def blk(b, carry):
            r0 = b * (3 * BJ)
            pltpu.async_copy(
                sig_hbm.at[idx_v.at[pl.ds(r0, 3 * BJ)]], rows_v, sem
            ).wait()

            def jbody(jj, c2):
                base = 3 * jj
                wtri = w_v[pl.ds(r0 + base, 16)]
                w0 = splat(wtri, jnp.zeros((16, 1), jnp.int32))
                w1 = splat(wtri, jnp.ones((16, 1), jnp.int32))
                w2 = splat(wtri, jnp.full((16, 1), 2, jnp.int32))
                for c in range(nch):
                    sl = pl.ds(c * 16, 16)
                    o_v[jj, sl] = (
                        w0 * rows_v[base, sl]
                        + w1 * rows_v[base + 1, sl]
                        + w2 * rows_v[base + 2, sl]
                    )
                return c2

            lax.fori_loop(0, BJ, jbody, 0)
            pltpu.sync_copy(o_v, out_hbm.at[pl.ds(jb + b * BJ, BJ)])
            return carry

        lax.fori_loop(0, NBLK, blk, 0)

    return kern


def _conv_body(T, x_ref, w_ref, b_ref, g_ref, be_ref, o_ref):
    res = (
        jnp.dot(x_ref[...], w_ref[...], preferred_element_type=jnp.float32)
        + b_ref[...]
    )
    res = jnp.where(res > 0, res, jnp.exp(jnp.minimum(res, 0.0)) - 1.0)
    m = res[:, :T]
    for rot in range(1, A):
        m = jnp.maximum(m, res[:, rot * T : (rot + 1) * T])
    o_ref[...] = g_ref[...] * m + be_ref[...]


def _conv_tc(x, wall, brep, gs, be):
    K = x.shape[1]
    T = wall.shape[1] // A
    return pl.pallas_call(
        functools.partial(_conv_body, T),
        grid=(VPAD // VB,),
        in_specs=[
            pl.BlockSpec((VB, K), lambda i: (i, 0)),
            pl.BlockSpec((K, A * T), lambda i: (0, 0)),
            pl.BlockSpec((1, A * T), lambda i: (0, 0)),
            pl.BlockSpec((1, T), lambda i: (0, 0)),
            pl.BlockSpec((1, T), lambda i: (0, 0)),
        ],
        out_specs=pl.BlockSpec((VB, T), lambda i: (i, 0)),
        out_shape=jax.ShapeDtypeStruct((VPAD, T), jnp.float32),
    )(x, wall, brep, gs, be)


def _dense_body(x_ref, w_ref, b_ref, o_ref):
    o_ref[...] = (
        jnp.dot(x_ref[...], w_ref[...], preferred_element_type=jnp.float32)
        + b_ref[...]
    )


def _dense_out(x, w, b):
    NB = 512
    ngrid = (V + NB - 1) // NB
    return pl.pallas_call(
        _dense_body,
        grid=(VPAD // 256, ngrid),
        in_specs=[
            pl.BlockSpec((256, 256), lambda i, j: (i, 0)),
            pl.BlockSpec((256, NB), lambda i, j: (0, j)),
            pl.BlockSpec((1, NB), lambda i, j: (0, j)),
        ],
        out_specs=pl.BlockSpec((256, NB), lambda i, j: (i, j)),
        out_shape=jax.ShapeDtypeStruct((V, V), jnp.float32),
    )(x, w, b.reshape(1, V))


def _make_wall(W):
    T = W.shape[0]
    cols = []
    for rot in range(A):
        Wr = jnp.roll(W, rot, axis=2)
        cols.append(Wr.transpose(1, 2, 3, 0).reshape(-1, T))
    return jnp.concatenate(cols, axis=1)


def _bn_scale(g):
    return (g / jnp.sqrt(1.0 + 1e-3)).reshape(1, -1)


def kernel(signal, bc, norm_mean, norm_var, W_d0, b_d0, g_d0, be_d0,
           W_d1, b_d1, g_d1, be_d1, W_m, b_m, g_m, be_m,
           W_u0, b_u0, g_u0, be_u0, W_u1, b_u1, g_u1, be_u1, W_out, b_out):
    idx = bc[..., 0].astype(jnp.int32).reshape(-1)
    wts = bc[..., 1].reshape(-1).astype(jnp.float32)
    npad = 3 * (JPAD - J)
    idxp = jnp.concatenate([idx, jnp.zeros((npad,), jnp.int32)])
    wp = jnp.concatenate([wts, jnp.zeros((npad,), jnp.float32)])

    def layer(table, wall, bias, g, be):
        d = table.shape[1]
        interp = _interp_kernel(d)(table, idxp, wp)
        x = interp.reshape(VPAD, R * A * d)
        brep = jnp.tile(bias, (A,)).reshape(1, -1)
        return _conv_tc(x, wall, brep, _bn_scale(g), be.reshape(1, -1))

    inv_sigma = 1.0 / jnp.sqrt(norm_var)
    DP0 = 16
    W0n = W_d0 * inv_sigma[None, None, None, :]
    W0n = jnp.pad(W0n, ((0, 0), (0, 0), (0, 0), (0, DP0 - 3)))
    b0 = b_d0 - jnp.einsum("c,trac->t", norm_mean * inv_sigma, W_d0)
    sig0 = jnp.pad(signal, ((0, 0), (0, DP0 - 3)))

    s0 = layer(sig0, _make_wall(W0n), b0, g_d0, be_d0)
    s1 = layer(s0, _make_wall(W_d1), b_d1, g_d1, be_d1)
    mid = layer(s1, _make_wall(W_m), b_m, g_m, be_m)
    u0 = layer(mid, _make_wall(W_u0), b_u0, g_u0, be_u0)
    u0c = jnp.concatenate([u0, s1], axis=1)
    u1 = layer(u0c, _make_wall(W_u1), b_u1, g_u1, be_u1)
    u1c = jnp.concatenate([u1, s0], axis=1)

    return _dense_out(u1c, W_out, b_out)


# speedup vs baseline: 5.3613x; 1.3423x over previous
import functools

import jax
import jax.numpy as jnp
from jax import lax
from jax.experimental import pallas as pl
from jax.experimental.pallas import tpu as pltpu
from jax.experimental.pallas import tpu_sc as plsc

V = 6890
R = 5
A = 8
VPAD = 6912
J = V * R * A
JPAD = VPAD * R * A
NC, NS = 2, 16
NW = NC * NS
JW = JPAD // NW
BJ = 48
NBLK = JW // BJ
VB = 128


@functools.cache
def _interp_kernel(d):
    mesh = plsc.VectorSubcoreMesh(
        core_axis_name="c", subcore_axis_name="s", num_cores=NC, num_subcores=NS
    )
    nch = d // 16

    @functools.partial(
        pl.kernel,
        out_type=jax.ShapeDtypeStruct((JPAD, d), jnp.float32),
        mesh=mesh,
        compiler_params=pltpu.CompilerParams(use_tc_tiling_on_sc=False),
        scratch_types=[
            pltpu.VMEM((3 * JW,), jnp.int32),
            pltpu.VMEM((3 * JW + 16,), jnp.float32),
            pltpu.VMEM((3 * BJ, d), jnp.float32),
            pltpu.VMEM((3 * BJ, d), jnp.float32),
            pltpu.VMEM((BJ, d), jnp.float32),
            pltpu.VMEM((BJ, d), jnp.float32),
            pltpu.SemaphoreType.DMA,
            pltpu.SemaphoreType.DMA,
            pltpu.SemaphoreType.DMA,
        ],
    )
    def kern(sig_hbm, idx_hbm, w_hbm, out_hbm,
             idx_v, w_v, rows0, rows1, o0, o1, gsem0, gsem1, osem):
        wid = lax.axis_index("s") * NC + lax.axis_index("c")
        jb = wid * JW
        rows = (rows0, rows1)
        gsem = (gsem0, gsem1)
        outs = (o0, o1)
        pltpu.sync_copy(idx_hbm.at[pl.ds(3 * jb, 3 * JW)], idx_v)
        pltpu.sync_copy(w_hbm.at[pl.ds(3 * jb, 3 * JW)], w_v.at[pl.ds(0, 3 * JW)])
        dn = lax.GatherDimensionNumbers(
            offset_dims=(), collapsed_slice_dims=(0,), start_index_map=(0,)
        )
        splat = functools.partial(
            lax.gather,
            dimension_numbers=dn,
            slice_sizes=(1,),
            mode=lax.GatherScatterMode.PROMISE_IN_BOUNDS,
        )

        def gth_start(b, buf, sem):
            return pltpu.async_copy(
                sig_hbm.at[idx_v.at[pl.ds(b * (3 * BJ), 3 * BJ)]], buf, sem
            )

        gth_start(0, rows0, gsem0)

        def pair(b0, carry):
            for par in (0, 1):
                b = b0 + par
                nxt = 1 - par

                @pl.when(b + 1 < NBLK)
                def _():
                    gth_start(b + 1, rows[nxt], gsem[nxt])

                pltpu.make_async_copy(
                    sig_hbm.at[idx_v.at[pl.ds(b * (3 * BJ), 3 * BJ)]],
                    rows[par], gsem[par],
                ).wait()
                rv = rows[par]
                ov = outs[par]

                def jbody(jj, c2):
                    base = 3 * jj
                    wtri = w_v[pl.ds(b * (3 * BJ) + base, 16)]
                    w0 = splat(wtri, jnp.zeros((16, 1), jnp.int32))
                    w1 = splat(wtri, jnp.ones((16, 1), jnp.int32))
                    w2 = splat(wtri, jnp.full((16, 1), 2, jnp.int32))
                    for c in range(nch):
                        sl = pl.ds(c * 16, 16)
                        ov[jj, sl] = (
                            w0 * rv[base, sl]
                            + w1 * rv[base + 1, sl]
                            + w2 * rv[base + 2, sl]
                        )
                    return c2

                lax.fori_loop(0, BJ, jbody, 0)

                @pl.when(b >= 1)
                def _():
                    pltpu.make_async_copy(
                        outs[nxt], out_hbm.at[pl.ds(jb + (b - 1) * BJ, BJ)], osem
                    ).wait()

                pltpu.async_copy(ov, out_hbm.at[pl.ds(jb + b * BJ, BJ)], osem)
            return carry

        lax.fori_loop(0, NBLK // 2, lambda i, c: pair(2 * i, c), 0)
        pltpu.make_async_copy(
            outs[1], out_hbm.at[pl.ds(jb + (NBLK - 1) * BJ, BJ)], osem
        ).wait()

    return kern


def _conv_body(T, x_ref, w_ref, b_ref, g_ref, be_ref, o_ref):
    res = (
        jnp.dot(x_ref[...], w_ref[...], preferred_element_type=jnp.float32)
        + b_ref[...]
    )
    res = jnp.where(res > 0, res, jnp.exp(jnp.minimum(res, 0.0)) - 1.0)
    m = res[:, :T]
    for rot in range(1, A):
        m = jnp.maximum(m, res[:, rot * T : (rot + 1) * T])
    o_ref[...] = g_ref[...] * m + be_ref[...]


def _conv_tc(x, wall, brep, gs, be):
    K = x.shape[1]
    T = wall.shape[1] // A
    return pl.pallas_call(
        functools.partial(_conv_body, T),
        grid=(VPAD // VB,),
        in_specs=[
            pl.BlockSpec((VB, K), lambda i: (i, 0)),
            pl.BlockSpec((K, A * T), lambda i: (0, 0)),
            pl.BlockSpec((1, A * T), lambda i: (0, 0)),
            pl.BlockSpec((1, T), lambda i: (0, 0)),
            pl.BlockSpec((1, T), lambda i: (0, 0)),
        ],
        out_specs=pl.BlockSpec((VB, T), lambda i: (i, 0)),
        out_shape=jax.ShapeDtypeStruct((VPAD, T), jnp.float32),
    )(x, wall, brep, gs, be)


def _dense_body(x_ref, w_ref, b_ref, o_ref):
    o_ref[...] = (
        jnp.dot(x_ref[...], w_ref[...], preferred_element_type=jnp.float32)
        + b_ref[...]
    )


def _dense_out(x, w, b):
    NB = 512
    ngrid = (V + NB - 1) // NB
    return pl.pallas_call(
        _dense_body,
        grid=(VPAD // 256, ngrid),
        in_specs=[
            pl.BlockSpec((256, 256), lambda i, j: (i, 0)),
            pl.BlockSpec((256, NB), lambda i, j: (0, j)),
            pl.BlockSpec((1, NB), lambda i, j: (0, j)),
        ],
        out_specs=pl.BlockSpec((256, NB), lambda i, j: (i, j)),
        out_shape=jax.ShapeDtypeStruct((V, V), jnp.float32),
    )(x, w, b.reshape(1, V))


def _make_wall(W):
    T = W.shape[0]
    cols = []
    for rot in range(A):
        Wr = jnp.roll(W, rot, axis=2)
        cols.append(Wr.transpose(1, 2, 3, 0).reshape(-1, T))
    return jnp.concatenate(cols, axis=1)


def _bn_scale(g):
    return (g / jnp.sqrt(1.0 + 1e-3)).reshape(1, -1)


def kernel(signal, bc, norm_mean, norm_var, W_d0, b_d0, g_d0, be_d0,
           W_d1, b_d1, g_d1, be_d1, W_m, b_m, g_m, be_m,
           W_u0, b_u0, g_u0, be_u0, W_u1, b_u1, g_u1, be_u1, W_out, b_out):
    idx = bc[..., 0].astype(jnp.int32).reshape(-1)
    wts = bc[..., 1].reshape(-1).astype(jnp.float32)
    npad = 3 * (JPAD - J)
    idxp = jnp.concatenate([idx, jnp.zeros((npad,), jnp.int32)])
    wp = jnp.concatenate([wts, jnp.zeros((npad,), jnp.float32)])

    def layer(table, wall, bias, g, be):
        d = table.shape[1]
        interp = _interp_kernel(d)(table, idxp, wp)
        x = interp.reshape(VPAD, R * A * d)
        brep = jnp.tile(bias, (A,)).reshape(1, -1)
        return _conv_tc(x, wall, brep, _bn_scale(g), be.reshape(1, -1))

    inv_sigma = 1.0 / jnp.sqrt(norm_var)
    DP0 = 16
    W0n = W_d0 * inv_sigma[None, None, None, :]
    W0n = jnp.pad(W0n, ((0, 0), (0, 0), (0, 0), (0, DP0 - 3)))
    b0 = b_d0 - jnp.einsum("c,trac->t", norm_mean * inv_sigma, W_d0)
    sig0 = jnp.pad(signal, ((0, 0), (0, DP0 - 3)))

    s0 = layer(sig0, _make_wall(W0n), b0, g_d0, be_d0)
    s1 = layer(s0, _make_wall(W_d1), b_d1, g_d1, be_d1)
    mid = layer(s1, _make_wall(W_m), b_m, g_m, be_m)
    u0 = layer(mid, _make_wall(W_u0), b_u0, g_u0, be_u0)
    u0c = jnp.concatenate([u0, s1], axis=1)
    u1 = layer(u0c, _make_wall(W_u1), b_u1, g_u1, be_u1)
    u1c = jnp.concatenate([u1, s0], axis=1)

    return _dense_out(u1c, W_out, b_out)


# speedup vs baseline: 5.7955x; 1.0810x over previous
import functools

import jax
import jax.numpy as jnp
from jax import lax
from jax.experimental import pallas as pl
from jax.experimental.pallas import tpu as pltpu
from jax.experimental.pallas import tpu_sc as plsc

V = 6890
R = 5
A = 8
VPAD = 6912
J = V * R * A
JPAD = VPAD * R * A
NC, NS = 2, 16
NW = NC * NS
JW = JPAD // NW
BJ = 48
NBLK = JW // BJ
VB = 128


@functools.cache
def _interp_kernel(d, bf16):
    mesh = plsc.VectorSubcoreMesh(
        core_axis_name="c", subcore_axis_name="s", num_cores=NC, num_subcores=NS
    )
    dt = jnp.bfloat16 if bf16 else jnp.float32

    @functools.partial(
        pl.kernel,
        out_type=jax.ShapeDtypeStruct((JPAD, d), dt),
        mesh=mesh,
        compiler_params=pltpu.CompilerParams(
            use_tc_tiling_on_sc=False, needs_layout_passes=False
        ),
        scratch_types=[
            pltpu.VMEM((3 * JW,), jnp.int32),
            pltpu.VMEM((3 * JW + 16,), jnp.float32),
            pltpu.VMEM((3 * BJ, d), dt),
            pltpu.VMEM((3 * BJ, d), dt),
            pltpu.VMEM((BJ, d), dt),
            pltpu.VMEM((BJ, d), dt),
            pltpu.SemaphoreType.DMA,
            pltpu.SemaphoreType.DMA,
            pltpu.SemaphoreType.DMA,
        ],
    )
    def kern(sig_hbm, idx_hbm, w_hbm, out_hbm,
             idx_v, w_v, rows0, rows1, o0, o1, gsem0, gsem1, osem):
        wid = lax.axis_index("s") * NC + lax.axis_index("c")
        jb = wid * JW
        rows = (rows0, rows1)
        gsem = (gsem0, gsem1)
        outs = (o0, o1)
        pltpu.sync_copy(idx_hbm.at[pl.ds(3 * jb, 3 * JW)], idx_v)
        pltpu.sync_copy(w_hbm.at[pl.ds(3 * jb, 3 * JW)], w_v.at[pl.ds(0, 3 * JW)])
        dn = lax.GatherDimensionNumbers(
            offset_dims=(), collapsed_slice_dims=(0,), start_index_map=(0,)
        )
        splat = functools.partial(
            lax.gather,
            dimension_numbers=dn,
            slice_sizes=(1,),
            mode=lax.GatherScatterMode.PROMISE_IN_BOUNDS,
        )

        def gth_start(b, buf, sem):
            return pltpu.async_copy(
                sig_hbm.at[idx_v.at[pl.ds(b * (3 * BJ), 3 * BJ)]], buf, sem
            )

        gth_start(0, rows0, gsem0)

        def pair(b0, carry):
            for par in (0, 1):
                b = b0 + par
                nxt = 1 - par

                @pl.when(b + 1 < NBLK)
                def _():
                    gth_start(b + 1, rows[nxt], gsem[nxt])

                pltpu.make_async_copy(
                    sig_hbm.at[idx_v.at[pl.ds(b * (3 * BJ), 3 * BJ)]],
                    rows[par], gsem[par],
                ).wait()
                rv = rows[par]
                ov = outs[par]

                def jbody(jj, c2):
                    base = 3 * jj
                    wtri = w_v[pl.ds(b * (3 * BJ) + base, 16)]
                    w0 = splat(wtri, jnp.zeros((16, 1), jnp.int32))
                    w1 = splat(wtri, jnp.ones((16, 1), jnp.int32))
                    w2 = splat(wtri, jnp.full((16, 1), 2, jnp.int32))
                    if bf16:
                        for c in range(d // 32):
                            sl = pl.ds(c * 32, 32)
                            a0, b0_ = plsc.unpack(
                                rv[base, sl], format=plsc.PackFormat.INTERLEAVED
                            )
                            a1, b1_ = plsc.unpack(
                                rv[base + 1, sl], format=plsc.PackFormat.INTERLEAVED
                            )
                            a2, b2_ = plsc.unpack(
                                rv[base + 2, sl], format=plsc.PackFormat.INTERLEAVED
                            )
                            oa = w0 * a0 + w1 * a1 + w2 * a2
                            ob = w0 * b0_ + w1 * b1_ + w2 * b2_
                            ov[jj, sl] = plsc.pack(
                                oa, ob, format=plsc.PackFormat.INTERLEAVED
                            )
                    else:
                        for c in range(d // 16):
                            sl = pl.ds(c * 16, 16)
                            ov[jj, sl] = (
                                w0 * rv[base, sl]
                                + w1 * rv[base + 1, sl]
                                + w2 * rv[base + 2, sl]
                            )
                    return c2

                lax.fori_loop(0, BJ, jbody, 0)

                @pl.when(b >= 1)
                def _():
                    pltpu.make_async_copy(
                        outs[nxt], out_hbm.at[pl.ds(jb + (b - 1) * BJ, BJ)], osem
                    ).wait()

                pltpu.async_copy(ov, out_hbm.at[pl.ds(jb + b * BJ, BJ)], osem)
            return carry

        lax.fori_loop(0, NBLK // 2, lambda i, c: pair(2 * i, c), 0)
        pltpu.make_async_copy(
            outs[1], out_hbm.at[pl.ds(jb + (NBLK - 1) * BJ, BJ)], osem
        ).wait()

    return kern


def _conv_body(T, x_ref, w_ref, b_ref, g_ref, be_ref, o_ref):
    res = (
        jnp.dot(x_ref[...], w_ref[...], preferred_element_type=jnp.float32)
        + b_ref[...]
    )
    res = jnp.where(res > 0, res, jnp.exp(jnp.minimum(res, 0.0)) - 1.0)
    m = res[:, :T]
    for rot in range(1, A):
        m = jnp.maximum(m, res[:, rot * T : (rot + 1) * T])
    o_ref[...] = (g_ref[...] * m + be_ref[...]).astype(o_ref.dtype)


def _conv_tc(x, wall, brep, gs, be):
    K = x.shape[1]
    T = wall.shape[1] // A
    return pl.pallas_call(
        functools.partial(_conv_body, T),
        grid=(VPAD // VB,),
        in_specs=[
            pl.BlockSpec((VB, K), lambda i: (i, 0)),
            pl.BlockSpec((K, A * T), lambda i: (0, 0)),
            pl.BlockSpec((1, A * T), lambda i: (0, 0)),
            pl.BlockSpec((1, T), lambda i: (0, 0)),
            pl.BlockSpec((1, T), lambda i: (0, 0)),
        ],
        out_specs=pl.BlockSpec((VB, T), lambda i: (i, 0)),
        out_shape=jax.ShapeDtypeStruct((VPAD, T), jnp.bfloat16),
    )(x, wall, brep, gs, be)


def _dense_body(x_ref, w_ref, b_ref, o_ref):
    o_ref[...] = (
        jnp.dot(x_ref[...], w_ref[...], preferred_element_type=jnp.float32)
        + b_ref[...]
    )


def _dense_out(x, w, b):
    NB = 512
    ngrid = (V + NB - 1) // NB
    return pl.pallas_call(
        _dense_body,
        grid=(VPAD // 256, ngrid),
        in_specs=[
            pl.BlockSpec((256, 256), lambda i, j: (i, 0)),
            pl.BlockSpec((256, NB), lambda i, j: (0, j)),
            pl.BlockSpec((1, NB), lambda i, j: (0, j)),
        ],
        out_specs=pl.BlockSpec((256, NB), lambda i, j: (i, j)),
        out_shape=jax.ShapeDtypeStruct((V, V), jnp.float32),
    )(x, w, b.reshape(1, V))


def _make_wall(W):
    T = W.shape[0]
    cols = []
    for rot in range(A):
        Wr = jnp.roll(W, rot, axis=2)
        cols.append(Wr.transpose(1, 2, 3, 0).reshape(-1, T))
    return jnp.concatenate(cols, axis=1)


def _bn_scale(g):
    return (g / jnp.sqrt(1.0 + 1e-3)).reshape(1, -1)


def kernel(signal, bc, norm_mean, norm_var, W_d0, b_d0, g_d0, be_d0,
           W_d1, b_d1, g_d1, be_d1, W_m, b_m, g_m, be_m,
           W_u0, b_u0, g_u0, be_u0, W_u1, b_u1, g_u1, be_u1, W_out, b_out):
    idx = bc[..., 0].astype(jnp.int32).reshape(-1)
    wts = bc[..., 1].reshape(-1).astype(jnp.float32)
    npad = 3 * (JPAD - J)
    idxp = jnp.concatenate([idx, jnp.zeros((npad,), jnp.int32)])
    wp = jnp.concatenate([wts, jnp.zeros((npad,), jnp.float32)])

    def layer(table, wall, bias, g, be):
        d = table.shape[1]
        bf16 = table.dtype == jnp.bfloat16
        interp = _interp_kernel(d, bf16)(table, idxp, wp)
        x = interp.reshape(VPAD, R * A * d)
        brep = jnp.tile(bias, (A,)).reshape(1, -1)
        wall = wall.astype(x.dtype)
        return _conv_tc(x, wall, brep, _bn_scale(g), be.reshape(1, -1))

    inv_sigma = 1.0 / jnp.sqrt(norm_var)
    DP0 = 16
    W0n = W_d0 * inv_sigma[None, None, None, :]
    W0n = jnp.pad(W0n, ((0, 0), (0, 0), (0, 0), (0, DP0 - 3)))
    b0 = b_d0 - jnp.einsum("c,trac->t", norm_mean * inv_sigma, W_d0)
    sig0 = jnp.pad(signal, ((0, 0), (0, DP0 - 3)))

    s0 = layer(sig0, _make_wall(W0n), b0, g_d0, be_d0)
    s1 = layer(s0, _make_wall(W_d1), b_d1, g_d1, be_d1)
    mid = layer(s1, _make_wall(W_m), b_m, g_m, be_m)
    u0 = layer(mid, _make_wall(W_u0), b_u0, g_u0, be_u0)
    u0c = jnp.concatenate([u0, s1], axis=1)
    u1 = layer(u0c, _make_wall(W_u1), b_u1, g_u1, be_u1)
    u1c = jnp.concatenate([u1, s0], axis=1)

    return _dense_out(u1c, W_out.astype(jnp.bfloat16), b_out)
